# baseline (speedup 1.0000x reference)
import functools

import jax
import jax.numpy as jnp
from jax import lax
from jax.experimental import pallas as pl
from jax.experimental.pallas import tpu as pltpu
from jax.experimental.pallas import tpu_sc as plsc

_N_EXPERT = 64
_TOP_K = 8
_TOKEN_BLOCK = 1024
_N_WORKERS = 32
_CHUNK = 256
_LANES = 16

_SORT8 = [
    (0, 1), (2, 3), (4, 5), (6, 7),
    (0, 2), (1, 3), (4, 6), (5, 7),
    (1, 2), (5, 6),
    (0, 4), (1, 5), (2, 6), (3, 7),
    (2, 4), (3, 5),
    (1, 2), (3, 4), (5, 6),
]
_BITONIC8 = [
    (0, 4), (1, 5), (2, 6), (3, 7),
    (0, 2), (1, 3), (4, 6), (5, 7),
    (0, 1), (2, 3), (4, 5), (6, 7),
]


def _ce(arr, i, j):
    a, b = arr[i], arr[j]
    arr[i] = jnp.maximum(a, b)
    arr[j] = jnp.minimum(a, b)


def _top8_sorted(keys):
    octs = []
    for o in range(8):
        oct_keys = keys[o * 8:(o + 1) * 8]
        for i, j in _SORT8:
            _ce(oct_keys, i, j)
        octs.append(oct_keys)
    while len(octs) > 1:
        merged = []
        for p in range(0, len(octs), 2):
            a, b = octs[p], octs[p + 1]
            t = [jnp.maximum(a[i], b[7 - i]) for i in range(8)]
            for i, j in _BITONIC8:
                _ce(t, i, j)
            merged.append(t)
        octs = merged
    return octs[0]


def _logits_block(x_ref, wt_ref, out_ref):
    x = x_ref[...]
    wt = wt_ref[...]
    logits = jax.lax.dot_general(
        x, wt, (((1,), (0,)), ((), ())), preferred_element_type=jnp.float32
    )
    out_ref[...] = logits.T.reshape(1, _N_EXPERT, _TOKEN_BLOCK)


def _route_body(logits3, probs_hbm, idx_hbm, lbuf, pbuf, ibuf):
    wid = lax.axis_index("s") * 2 + lax.axis_index("c")
    lane = lax.iota(jnp.int32, _LANES)
    zero = jnp.zeros((_LANES,), jnp.float32)
    n_blocks = logits3.shape[0]
    tokens_per_worker = n_blocks * _TOKEN_BLOCK // _N_WORKERS

    def chunk_body(c, carry):
        base = wid * tokens_per_worker + c * _CHUNK
        blk = base // _TOKEN_BLOCK
        off = base % _TOKEN_BLOCK
        pltpu.sync_copy(logits3.at[blk, :, pl.ds(off, _CHUNK)], lbuf)

        def group_body(g, inner):
            t0 = g * _LANES
            keys = []
            for e in range(_N_EXPERT):
                v = lbuf[e, pl.ds(t0, _LANES)]
                b = plsc.bitcast(v, jnp.int32)
                k = jnp.where(b < 0, b ^ jnp.int32(0x7FFFFFFF), b)
                keys.append((k | jnp.int32(63)) ^ jnp.int32(e))
            top = _top8_sorted(keys)

            tok = t0 + lane
            for i in range(_LANES):
                for j in range(_N_EXPERT // _LANES):
                    pbuf[t0 + i, pl.ds(j * _LANES, _LANES)] = zero

            experts = [63 - (top[k] & jnp.int32(63)) for k in range(_TOP_K)]
            vals = [
                plsc.load_gather(lbuf, [experts[k], tok])
                for k in range(_TOP_K)
            ]
            m0 = vals[0]
            exps = [jnp.exp(vals[k] - m0) for k in range(_TOP_K)]
            denom = exps[0]
            for k in range(1, _TOP_K):
                denom = denom + exps[k]
            inv = 1.0 / denom
            for k in range(_TOP_K):
                plsc.store_scatter(pbuf, [tok, experts[k]], exps[k] * inv)
                plsc.store_scatter(
                    ibuf,
                    [tok, jnp.full((_LANES,), k, jnp.int32)],
                    experts[k],
                )
            return inner

        lax.fori_loop(0, _CHUNK // _LANES, group_body, 0)
        pltpu.sync_copy(pbuf, probs_hbm.at[pl.ds(base, _CHUNK)])
        pltpu.sync_copy(ibuf, idx_hbm.at[pl.ds(base, _CHUNK)])
        return carry

    lax.fori_loop(0, tokens_per_worker // _CHUNK, chunk_body, 0)


_N_SLICES = 4


def _matmul_slice(x_slice, wt):
    n_tok, d = x_slice.shape
    return pl.pallas_call(
        _logits_block,
        grid=(n_tok // _TOKEN_BLOCK,),
        in_specs=[
            pl.BlockSpec((_TOKEN_BLOCK, d), lambda i: (i, 0)),
            pl.BlockSpec((d, _N_EXPERT), lambda i: (0, 0)),
        ],
        out_specs=pl.BlockSpec(
            (1, _N_EXPERT, _TOKEN_BLOCK), lambda i: (i, 0, 0)
        ),
        out_shape=jax.ShapeDtypeStruct(
            (n_tok // _TOKEN_BLOCK, _N_EXPERT, _TOKEN_BLOCK), jnp.float32
        ),
        compiler_params=pltpu.CompilerParams(
            dimension_semantics=("parallel",)
        ),
    )(x_slice, wt)


def _route_slice(logits3):
    n_tok = logits3.shape[0] * _TOKEN_BLOCK
    route = functools.partial(
        pl.kernel,
        out_type=[
            jax.ShapeDtypeStruct((n_tok, _N_EXPERT), jnp.float32),
            jax.ShapeDtypeStruct((n_tok, _TOP_K), jnp.int32),
        ],
        mesh=plsc.VectorSubcoreMesh(core_axis_name="c", subcore_axis_name="s"),
        compiler_params=pltpu.CompilerParams(needs_layout_passes=False),
        scratch_types=[
            pltpu.VMEM((_N_EXPERT, _CHUNK), jnp.float32),
            pltpu.VMEM((_CHUNK, _N_EXPERT), jnp.float32),
            pltpu.VMEM((_CHUNK, _TOP_K), jnp.int32),
        ],
    )(_route_body)
    return route(logits3)


def kernel(x, W):
    n_tokens, d = x.shape
    wt = W.T
    per_slice = n_tokens // _N_SLICES
    probs_parts, idx_parts = [], []
    for s in range(_N_SLICES):
        xs = x[s * per_slice:(s + 1) * per_slice]
        logits3 = _matmul_slice(xs, wt)
        p, i = _route_slice(logits3)
        probs_parts.append(p)
        idx_parts.append(i)
    return (
        jnp.concatenate(probs_parts, axis=0),
        jnp.concatenate(idx_parts, axis=0),
    )


# speedup vs baseline: 2.3505x; 2.3505x over previous
import functools

import jax
import jax.numpy as jnp
from jax import lax
from jax.experimental import pallas as pl
from jax.experimental.pallas import tpu as pltpu
from jax.experimental.pallas import tpu_sc as plsc

_N_EXPERT = 64
_TOP_K = 8
_TOKEN_BLOCK = 1024
_N_WORKERS = 32
_CHUNK = 256
_LANES = 16

_SORT8 = [
    (0, 1), (2, 3), (4, 5), (6, 7),
    (0, 2), (1, 3), (4, 6), (5, 7),
    (1, 2), (5, 6),
    (0, 4), (1, 5), (2, 6), (3, 7),
    (2, 4), (3, 5),
    (1, 2), (3, 4), (5, 6),
]
_BITONIC8 = [
    (0, 4), (1, 5), (2, 6), (3, 7),
    (0, 2), (1, 3), (4, 6), (5, 7),
    (0, 1), (2, 3), (4, 5), (6, 7),
]


def _ce(arr, i, j):
    a, b = arr[i], arr[j]
    arr[i] = jnp.maximum(a, b)
    arr[j] = jnp.minimum(a, b)


def _top8_sorted(keys):
    octs = []
    for o in range(8):
        oct_keys = keys[o * 8:(o + 1) * 8]
        for i, j in _SORT8:
            _ce(oct_keys, i, j)
        octs.append(oct_keys)
    while len(octs) > 1:
        merged = []
        for p in range(0, len(octs), 2):
            a, b = octs[p], octs[p + 1]
            t = [jnp.maximum(a[i], b[7 - i]) for i in range(8)]
            for i, j in _BITONIC8:
                _ce(t, i, j)
            merged.append(t)
        octs = merged
    return octs[0]


def _logits_block(x_ref, wt_ref, out_ref):
    x = x_ref[...]
    wt = wt_ref[...]
    logits = jax.lax.dot_general(
        x, wt, (((1,), (0,)), ((), ())), preferred_element_type=jnp.float32
    )
    out_ref[...] = logits.T.reshape(1, _N_EXPERT, _TOKEN_BLOCK)


def _route_body(logits3, probs_hbm, idx_hbm, lbuf, pbuf, ibuf):
    wid = lax.axis_index("s") * 2 + lax.axis_index("c")
    lane = lax.iota(jnp.int32, _LANES)
    zero = jnp.zeros((_LANES,), jnp.float32)
    n_blocks = logits3.shape[0]
    tokens_per_worker = n_blocks * _TOKEN_BLOCK // _N_WORKERS

    def chunk_body(c, carry):
        base = wid * tokens_per_worker + c * _CHUNK
        blk = base // _TOKEN_BLOCK
        off = base % _TOKEN_BLOCK
        pltpu.sync_copy(logits3.at[blk, :, pl.ds(off, _CHUNK)], lbuf)

        def group_body(g, inner):
            t0 = g * _LANES
            keys = []
            for e in range(_N_EXPERT):
                v = lbuf[e, pl.ds(t0, _LANES)]
                b = plsc.bitcast(v, jnp.int32)
                k = jnp.where(b < 0, b ^ jnp.int32(0x7FFFFFFF), b)
                keys.append((k | jnp.int32(63)) ^ jnp.int32(e))
            top = _top8_sorted(keys)

            tok = t0 + lane
            for i in range(_LANES):
                for j in range(_N_EXPERT // _LANES):
                    pbuf[t0 + i, pl.ds(j * _LANES, _LANES)] = zero

            experts = [63 - (top[k] & jnp.int32(63)) for k in range(_TOP_K)]
            vals = [
                plsc.load_gather(lbuf, [experts[k], tok])
                for k in range(_TOP_K)
            ]
            m0 = vals[0]
            exps = [jnp.exp(vals[k] - m0) for k in range(_TOP_K)]
            denom = exps[0]
            for k in range(1, _TOP_K):
                denom = denom + exps[k]
            inv = 1.0 / denom
            for k in range(_TOP_K):
                plsc.store_scatter(pbuf, [tok, experts[k]], exps[k] * inv)
                plsc.store_scatter(
                    ibuf,
                    [tok, jnp.full((_LANES,), k, jnp.int32)],
                    experts[k],
                )
            return inner

        lax.fori_loop(0, _CHUNK // _LANES, group_body, 0)
        pltpu.sync_copy(pbuf, probs_hbm.at[pl.ds(base, _CHUNK)])
        pltpu.sync_copy(ibuf, idx_hbm.at[pl.ds(base, _CHUNK)])
        return carry

    lax.fori_loop(0, tokens_per_worker // _CHUNK, chunk_body, 0)


_N_SLICES = 4


def _matmul_slice(x, wt, block0, n_blocks):
    d = x.shape[1]
    return pl.pallas_call(
        _logits_block,
        grid=(n_blocks,),
        in_specs=[
            pl.BlockSpec((_TOKEN_BLOCK, d), lambda i: (i + block0, 0)),
            pl.BlockSpec((d, _N_EXPERT), lambda i: (0, 0)),
        ],
        out_specs=pl.BlockSpec(
            (1, _N_EXPERT, _TOKEN_BLOCK), lambda i: (i, 0, 0)
        ),
        out_shape=jax.ShapeDtypeStruct(
            (n_blocks, _N_EXPERT, _TOKEN_BLOCK), jnp.float32
        ),
        compiler_params=pltpu.CompilerParams(
            dimension_semantics=("parallel",)
        ),
    )(x, wt)


def _route_slice(logits3):
    n_tok = logits3.shape[0] * _TOKEN_BLOCK
    route = functools.partial(
        pl.kernel,
        out_type=[
            jax.ShapeDtypeStruct((n_tok, _N_EXPERT), jnp.float32),
            jax.ShapeDtypeStruct((n_tok, _TOP_K), jnp.int32),
        ],
        mesh=plsc.VectorSubcoreMesh(core_axis_name="c", subcore_axis_name="s"),
        compiler_params=pltpu.CompilerParams(needs_layout_passes=False),
        scratch_types=[
            pltpu.VMEM((_N_EXPERT, _CHUNK), jnp.float32),
            pltpu.VMEM((_CHUNK, _N_EXPERT), jnp.float32),
            pltpu.VMEM((_CHUNK, _TOP_K), jnp.int32),
        ],
    )(_route_body)
    return route(logits3)


def kernel(x, W):
    n_tokens, d = x.shape
    wt = W.T
    blocks_per_slice = n_tokens // _TOKEN_BLOCK // _N_SLICES
    probs_parts, idx_parts = [], []
    for s in range(_N_SLICES):
        logits3 = _matmul_slice(x, wt, s * blocks_per_slice, blocks_per_slice)
        p, i = _route_slice(logits3)
        probs_parts.append(p)
        idx_parts.append(i)
    return (
        jnp.concatenate(probs_parts, axis=0),
        jnp.concatenate(idx_parts, axis=0),
    )


# speedup vs baseline: 2.3573x; 1.0029x over previous
import functools

import jax
import jax.numpy as jnp
from jax import lax
from jax.experimental import pallas as pl
from jax.experimental.pallas import tpu as pltpu
from jax.experimental.pallas import tpu_sc as plsc

_N_EXPERT = 64
_TOP_K = 8
_TOKEN_BLOCK = 1024
_N_WORKERS = 32
_CHUNK = 256
_LANES = 16

_SORT8 = [
    (0, 1), (2, 3), (4, 5), (6, 7),
    (0, 2), (1, 3), (4, 6), (5, 7),
    (1, 2), (5, 6),
    (0, 4), (1, 5), (2, 6), (3, 7),
    (2, 4), (3, 5),
    (1, 2), (3, 4), (5, 6),
]
_BITONIC8 = [
    (0, 4), (1, 5), (2, 6), (3, 7),
    (0, 2), (1, 3), (4, 6), (5, 7),
    (0, 1), (2, 3), (4, 5), (6, 7),
]


def _ce(arr, i, j):
    a, b = arr[i], arr[j]
    arr[i] = jnp.maximum(a, b)
    arr[j] = jnp.minimum(a, b)


def _top8_sorted(keys):
    octs = []
    for o in range(8):
        oct_keys = keys[o * 8:(o + 1) * 8]
        for i, j in _SORT8:
            _ce(oct_keys, i, j)
        octs.append(oct_keys)
    while len(octs) > 1:
        merged = []
        for p in range(0, len(octs), 2):
            a, b = octs[p], octs[p + 1]
            t = [jnp.maximum(a[i], b[7 - i]) for i in range(8)]
            for i, j in _BITONIC8:
                _ce(t, i, j)
            merged.append(t)
        octs = merged
    return octs[0]


def _logits_block(x_ref, wt_ref, out_ref):
    x = x_ref[...]
    wt = wt_ref[...]
    logits = jax.lax.dot_general(
        x, wt, (((1,), (0,)), ((), ())), preferred_element_type=jnp.float32
    )
    out_ref[...] = logits.T.reshape(1, _N_EXPERT, _TOKEN_BLOCK)


def _route_body(logits3, probs_hbm, idx_hbm, lbuf, pbuf, ibuf):
    wid = lax.axis_index("s") * 2 + lax.axis_index("c")
    lane = lax.iota(jnp.int32, _LANES)
    zero = jnp.zeros((_LANES,), jnp.float32)
    n_blocks = logits3.shape[0]
    tokens_per_worker = n_blocks * _TOKEN_BLOCK // _N_WORKERS

    def chunk_body(c, carry):
        base = wid * tokens_per_worker + c * _CHUNK
        blk = base // _TOKEN_BLOCK
        off = base % _TOKEN_BLOCK
        pltpu.sync_copy(logits3.at[blk, :, pl.ds(off, _CHUNK)], lbuf)

        def group_body(g, inner):
            t0 = g * _LANES
            keys = []
            for e in range(_N_EXPERT):
                v = lbuf[e, pl.ds(t0, _LANES)]
                b = plsc.bitcast(v, jnp.int32)
                k = jnp.where(b < 0, b ^ jnp.int32(0x7FFFFFFF), b)
                keys.append((k | jnp.int32(63)) ^ jnp.int32(e))
            top = _top8_sorted(keys)

            tok = t0 + lane
            for i in range(_LANES):
                for j in range(_N_EXPERT // _LANES):
                    pbuf[t0 + i, pl.ds(j * _LANES, _LANES)] = zero

            experts = [63 - (top[k] & jnp.int32(63)) for k in range(_TOP_K)]
            vals = [
                plsc.load_gather(lbuf, [experts[k], tok])
                for k in range(_TOP_K)
            ]
            m0 = vals[0]
            exps = [jnp.exp(vals[k] - m0) for k in range(_TOP_K)]
            denom = exps[0]
            for k in range(1, _TOP_K):
                denom = denom + exps[k]
            inv = 1.0 / denom
            for k in range(_TOP_K):
                plsc.store_scatter(pbuf, [tok, experts[k]], exps[k] * inv)
                plsc.store_scatter(
                    ibuf,
                    [tok, jnp.full((_LANES,), k, jnp.int32)],
                    experts[k],
                )
            return inner

        lax.fori_loop(0, _CHUNK // _LANES, group_body, 0)
        pltpu.sync_copy(pbuf, probs_hbm.at[pl.ds(base, _CHUNK)])
        pltpu.sync_copy(ibuf, idx_hbm.at[pl.ds(base, _CHUNK)])
        return carry

    lax.fori_loop(0, tokens_per_worker // _CHUNK, chunk_body, 0)


_N_SLICES = 2


def _matmul_slice(x, wt, block0, n_blocks):
    d = x.shape[1]
    return pl.pallas_call(
        _logits_block,
        grid=(n_blocks,),
        in_specs=[
            pl.BlockSpec((_TOKEN_BLOCK, d), lambda i: (i + block0, 0)),
            pl.BlockSpec((d, _N_EXPERT), lambda i: (0, 0)),
        ],
        out_specs=pl.BlockSpec(
            (1, _N_EXPERT, _TOKEN_BLOCK), lambda i: (i, 0, 0)
        ),
        out_shape=jax.ShapeDtypeStruct(
            (n_blocks, _N_EXPERT, _TOKEN_BLOCK), jnp.float32
        ),
        compiler_params=pltpu.CompilerParams(
            dimension_semantics=("parallel",)
        ),
    )(x, wt)


def _route_slice(logits3):
    n_tok = logits3.shape[0] * _TOKEN_BLOCK
    route = functools.partial(
        pl.kernel,
        out_type=[
            jax.ShapeDtypeStruct((n_tok, _N_EXPERT), jnp.float32),
            jax.ShapeDtypeStruct((n_tok, _TOP_K), jnp.int32),
        ],
        mesh=plsc.VectorSubcoreMesh(core_axis_name="c", subcore_axis_name="s"),
        compiler_params=pltpu.CompilerParams(needs_layout_passes=False),
        scratch_types=[
            pltpu.VMEM((_N_EXPERT, _CHUNK), jnp.float32),
            pltpu.VMEM((_CHUNK, _N_EXPERT), jnp.float32),
            pltpu.VMEM((_CHUNK, _TOP_K), jnp.int32),
        ],
    )(_route_body)
    return route(logits3)


def kernel(x, W):
    n_tokens, d = x.shape
    wt = W.T
    blocks_per_slice = n_tokens // _TOKEN_BLOCK // _N_SLICES
    probs_parts, idx_parts = [], []
    for s in range(_N_SLICES):
        logits3 = _matmul_slice(x, wt, s * blocks_per_slice, blocks_per_slice)
        p, i = _route_slice(logits3)
        probs_parts.append(p)
        idx_parts.append(i)
    return (
        jnp.concatenate(probs_parts, axis=0),
        jnp.concatenate(idx_parts, axis=0),
    )


# speedup vs baseline: 2.4285x; 1.0302x over previous
import functools

import jax
import jax.numpy as jnp
from jax import lax
from jax.experimental import pallas as pl
from jax.experimental.pallas import tpu as pltpu
from jax.experimental.pallas import tpu_sc as plsc

_N_EXPERT = 64
_TOP_K = 8
_TOKEN_BLOCK = 1024
_N_WORKERS = 32
_CHUNK = 256
_LANES = 16

_SORT8 = [
    (0, 1), (2, 3), (4, 5), (6, 7),
    (0, 2), (1, 3), (4, 6), (5, 7),
    (1, 2), (5, 6),
    (0, 4), (1, 5), (2, 6), (3, 7),
    (2, 4), (3, 5),
    (1, 2), (3, 4), (5, 6),
]
_BITONIC8 = [
    (0, 4), (1, 5), (2, 6), (3, 7),
    (0, 2), (1, 3), (4, 6), (5, 7),
    (0, 1), (2, 3), (4, 5), (6, 7),
]


def _ce(arr, i, j):
    a, b = arr[i], arr[j]
    arr[i] = jnp.maximum(a, b)
    arr[j] = jnp.minimum(a, b)


def _top8_sorted(keys):
    octs = []
    for o in range(8):
        oct_keys = keys[o * 8:(o + 1) * 8]
        for i, j in _SORT8:
            _ce(oct_keys, i, j)
        octs.append(oct_keys)
    while len(octs) > 1:
        merged = []
        for p in range(0, len(octs), 2):
            a, b = octs[p], octs[p + 1]
            t = [jnp.maximum(a[i], b[7 - i]) for i in range(8)]
            for i, j in _BITONIC8:
                _ce(t, i, j)
            merged.append(t)
        octs = merged
    return octs[0]


def _logits_block(x_ref, wt_ref, out_ref):
    x = x_ref[...]
    wt = wt_ref[...]
    logits = jax.lax.dot_general(
        x, wt, (((1,), (0,)), ((), ())), preferred_element_type=jnp.float32
    )
    out_ref[...] = logits.T.reshape(1, _N_EXPERT, _TOKEN_BLOCK)


def _route_body(logits3, probs_hbm, idx_hbm, lbuf, pbuf, ibuf,
                sem_in0, sem_in1):
    wid = lax.axis_index("s") * 2 + lax.axis_index("c")
    lane = lax.iota(jnp.int32, _LANES)
    zero = jnp.zeros((_LANES,), jnp.float32)
    n_blocks = logits3.shape[0]
    tokens_per_worker = n_blocks * _TOKEN_BLOCK // _N_WORKERS
    n_chunks = tokens_per_worker // _CHUNK
    sems_in = (sem_in0, sem_in1)

    def start_in(c):
        base = wid * tokens_per_worker + c * _CHUNK
        blk = base // _TOKEN_BLOCK
        off = base % _TOKEN_BLOCK
        return pltpu.async_copy(
            logits3.at[blk, :, pl.ds(off, _CHUNK)],
            lbuf.at[c % 2],
            sems_in[c % 2],
        )

    def process_chunk(c):
        lb = lbuf.at[c % 2]
        pb = pbuf
        ib = ibuf

        def group_body(g, inner):
            t0 = g * _LANES
            keys = []
            for e in range(_N_EXPERT):
                v = lb[e, pl.ds(t0, _LANES)]
                b = plsc.bitcast(v, jnp.int32)
                k = jnp.where(b < 0, b ^ jnp.int32(0x7FFFFFFF), b)
                keys.append((k | jnp.int32(63)) ^ jnp.int32(e))
            top = _top8_sorted(keys)

            tok = t0 + lane
            for i in range(_LANES):
                for j in range(_N_EXPERT // _LANES):
                    pb[t0 + i, pl.ds(j * _LANES, _LANES)] = zero

            experts = [63 - (top[k] & jnp.int32(63)) for k in range(_TOP_K)]
            vals = [
                plsc.load_gather(lb, [experts[k], tok])
                for k in range(_TOP_K)
            ]
            m0 = vals[0]
            exps = [jnp.exp(vals[k] - m0) for k in range(_TOP_K)]
            denom = exps[0]
            for k in range(1, _TOP_K):
                denom = denom + exps[k]
            inv = 1.0 / denom
            for k in range(_TOP_K):
                plsc.store_scatter(pb, [tok, experts[k]], exps[k] * inv)
                plsc.store_scatter(
                    ib,
                    [tok, jnp.full((_LANES,), k, jnp.int32)],
                    experts[k],
                )
            return inner

        lax.fori_loop(0, _CHUNK // _LANES, group_body, 0)

    def copy_out(c):
        base = wid * tokens_per_worker + c * _CHUNK
        pltpu.sync_copy(pbuf, probs_hbm.at[pl.ds(base, _CHUNK)])
        pltpu.sync_copy(ibuf, idx_hbm.at[pl.ds(base, _CHUNK)])

    in_handles = [None] * n_chunks
    in_handles[0] = start_in(0)
    for c in range(n_chunks):
        if c + 1 < n_chunks:
            in_handles[c + 1] = start_in(c + 1)
        in_handles[c].wait()
        process_chunk(c)
        copy_out(c)


_N_SLICES = 1


def _matmul_slice(x, wt, block0, n_blocks):
    d = x.shape[1]
    return pl.pallas_call(
        _logits_block,
        grid=(n_blocks,),
        in_specs=[
            pl.BlockSpec((_TOKEN_BLOCK, d), lambda i: (i + block0, 0)),
            pl.BlockSpec((d, _N_EXPERT), lambda i: (0, 0)),
        ],
        out_specs=pl.BlockSpec(
            (1, _N_EXPERT, _TOKEN_BLOCK), lambda i: (i, 0, 0)
        ),
        out_shape=jax.ShapeDtypeStruct(
            (n_blocks, _N_EXPERT, _TOKEN_BLOCK), jnp.float32
        ),
        compiler_params=pltpu.CompilerParams(
            dimension_semantics=("parallel",)
        ),
    )(x, wt)


def _route_slice(logits3):
    n_tok = logits3.shape[0] * _TOKEN_BLOCK
    route = functools.partial(
        pl.kernel,
        out_type=[
            jax.ShapeDtypeStruct((n_tok, _N_EXPERT), jnp.float32),
            jax.ShapeDtypeStruct((n_tok, _TOP_K), jnp.int32),
        ],
        mesh=plsc.VectorSubcoreMesh(core_axis_name="c", subcore_axis_name="s"),
        compiler_params=pltpu.CompilerParams(needs_layout_passes=False),
        scratch_types=[
            pltpu.VMEM((2, _N_EXPERT, _CHUNK), jnp.float32),
            pltpu.VMEM((_CHUNK, _N_EXPERT), jnp.float32),
            pltpu.VMEM((_CHUNK, _TOP_K), jnp.int32),
            pltpu.SemaphoreType.DMA,
            pltpu.SemaphoreType.DMA,
        ],
    )(_route_body)
    return route(logits3)


def kernel(x, W):
    n_tokens, d = x.shape
    wt = W.T
    blocks_per_slice = n_tokens // _TOKEN_BLOCK // _N_SLICES
    probs_parts, idx_parts = [], []
    for s in range(_N_SLICES):
        logits3 = _matmul_slice(x, wt, s * blocks_per_slice, blocks_per_slice)
        p, i = _route_slice(logits3)
        probs_parts.append(p)
        idx_parts.append(i)
    if _N_SLICES == 1:
        return (probs_parts[0], idx_parts[0])
    return (
        jnp.concatenate(probs_parts, axis=0),
        jnp.concatenate(idx_parts, axis=0),
    )


# speedup vs baseline: 2.4346x; 1.0025x over previous
import functools

import jax
import jax.numpy as jnp
from jax import lax
from jax.experimental import pallas as pl
from jax.experimental.pallas import tpu as pltpu
from jax.experimental.pallas import tpu_sc as plsc

_N_EXPERT = 64
_TOP_K = 8
_TOKEN_BLOCK = 1024
_N_WORKERS = 32
_CHUNK = 256
_LANES = 16

_SORT8 = [
    (0, 1), (2, 3), (4, 5), (6, 7),
    (0, 2), (1, 3), (4, 6), (5, 7),
    (1, 2), (5, 6),
    (0, 4), (1, 5), (2, 6), (3, 7),
    (2, 4), (3, 5),
    (1, 2), (3, 4), (5, 6),
]
_BITONIC8 = [
    (0, 4), (1, 5), (2, 6), (3, 7),
    (0, 2), (1, 3), (4, 6), (5, 7),
    (0, 1), (2, 3), (4, 5), (6, 7),
]


def _ce(arr, i, j):
    a, b = arr[i], arr[j]
    arr[i] = jnp.maximum(a, b)
    arr[j] = jnp.minimum(a, b)


def _top8_sorted(keys):
    octs = []
    for o in range(8):
        oct_keys = keys[o * 8:(o + 1) * 8]
        for i, j in _SORT8:
            _ce(oct_keys, i, j)
        octs.append(oct_keys)
    while len(octs) > 1:
        merged = []
        for p in range(0, len(octs), 2):
            a, b = octs[p], octs[p + 1]
            t = [jnp.maximum(a[i], b[7 - i]) for i in range(8)]
            for i, j in _BITONIC8:
                _ce(t, i, j)
            merged.append(t)
        octs = merged
    return octs[0]


def _logits_block(x_ref, wt_ref, out_ref):
    x = x_ref[...]
    wt = wt_ref[...]
    logits = jax.lax.dot_general(
        x, wt, (((1,), (0,)), ((), ())), preferred_element_type=jnp.float32
    )
    out_ref[...] = logits.T


def _route_body(logits3, probs_hbm, idx_hbm, lbuf, pbuf, ibuf,
                sem_in0, sem_in1):
    wid = lax.axis_index("s") * 2 + lax.axis_index("c")
    lane = lax.iota(jnp.int32, _LANES)
    zero = jnp.zeros((_LANES,), jnp.float32)
    tokens_per_worker = logits3.shape[1] // _N_WORKERS
    n_chunks = tokens_per_worker // _CHUNK
    sems_in = (sem_in0, sem_in1)

    def start_in(c):
        base = wid * tokens_per_worker + c * _CHUNK
        return pltpu.async_copy(
            logits3.at[:, pl.ds(base, _CHUNK)],
            lbuf.at[c % 2],
            sems_in[c % 2],
        )

    def process_chunk(c):
        lb = lbuf.at[c % 2]
        pb = pbuf
        ib = ibuf

        def group_body(g, inner):
            t0 = g * _LANES
            keys = []
            for e in range(_N_EXPERT):
                v = lb[e, pl.ds(t0, _LANES)]
                b = plsc.bitcast(v, jnp.int32)
                k = jnp.where(b < 0, b ^ jnp.int32(0x7FFFFFFF), b)
                keys.append((k | jnp.int32(63)) ^ jnp.int32(e))
            top = _top8_sorted(keys)

            tok = t0 + lane
            for i in range(_LANES):
                for j in range(_N_EXPERT // _LANES):
                    pb[t0 + i, pl.ds(j * _LANES, _LANES)] = zero

            experts = [63 - (top[k] & jnp.int32(63)) for k in range(_TOP_K)]
            vals = [
                plsc.load_gather(lb, [experts[k], tok])
                for k in range(_TOP_K)
            ]
            m0 = vals[0]
            exps = [jnp.exp(vals[k] - m0) for k in range(_TOP_K)]
            denom = exps[0]
            for k in range(1, _TOP_K):
                denom = denom + exps[k]
            inv = 1.0 / denom
            for k in range(_TOP_K):
                plsc.store_scatter(pb, [tok, experts[k]], exps[k] * inv)
                plsc.store_scatter(
                    ib,
                    [tok, jnp.full((_LANES,), k, jnp.int32)],
                    experts[k],
                )
            return inner

        lax.fori_loop(0, _CHUNK // _LANES, group_body, 0)

    def copy_out(c):
        base = wid * tokens_per_worker + c * _CHUNK
        pltpu.sync_copy(pbuf, probs_hbm.at[pl.ds(base, _CHUNK)])
        pltpu.sync_copy(ibuf, idx_hbm.at[pl.ds(base, _CHUNK)])

    in_handles = [None] * n_chunks
    in_handles[0] = start_in(0)
    for c in range(n_chunks):
        if c + 1 < n_chunks:
            in_handles[c + 1] = start_in(c + 1)
        in_handles[c].wait()
        process_chunk(c)
        copy_out(c)


_N_SLICES = 1


def _matmul_slice(x, wt, block0, n_blocks):
    d = x.shape[1]
    return pl.pallas_call(
        _logits_block,
        grid=(n_blocks,),
        in_specs=[
            pl.BlockSpec((_TOKEN_BLOCK, d), lambda i: (i + block0, 0)),
            pl.BlockSpec((d, _N_EXPERT), lambda i: (0, 0)),
        ],
        out_specs=pl.BlockSpec(
            (_N_EXPERT, _TOKEN_BLOCK), lambda i: (0, i + block0)
        ),
        out_shape=jax.ShapeDtypeStruct(
            (_N_EXPERT, n_blocks * _TOKEN_BLOCK), jnp.float32
        ),
        compiler_params=pltpu.CompilerParams(
            dimension_semantics=("parallel",)
        ),
    )(x, wt)


def _route_slice(logits3):
    n_tok = logits3.shape[1]
    route = functools.partial(
        pl.kernel,
        out_type=[
            jax.ShapeDtypeStruct((n_tok, _N_EXPERT), jnp.float32),
            jax.ShapeDtypeStruct((n_tok, _TOP_K), jnp.int32),
        ],
        mesh=plsc.VectorSubcoreMesh(core_axis_name="c", subcore_axis_name="s"),
        compiler_params=pltpu.CompilerParams(needs_layout_passes=False),
        scratch_types=[
            pltpu.VMEM((2, _N_EXPERT, _CHUNK), jnp.float32),
            pltpu.VMEM((_CHUNK, _N_EXPERT), jnp.float32),
            pltpu.VMEM((_CHUNK, _TOP_K), jnp.int32),
            pltpu.SemaphoreType.DMA,
            pltpu.SemaphoreType.DMA,
        ],
    )(_route_body)
    return route(logits3)


def kernel(x, W):
    n_tokens, d = x.shape
    wt = W.T
    blocks_per_slice = n_tokens // _TOKEN_BLOCK // _N_SLICES
    probs_parts, idx_parts = [], []
    for s in range(_N_SLICES):
        logits3 = _matmul_slice(x, wt, s * blocks_per_slice, blocks_per_slice)
        p, i = _route_slice(logits3)
        probs_parts.append(p)
        idx_parts.append(i)
    if _N_SLICES == 1:
        return (probs_parts[0], idx_parts[0])
    return (
        jnp.concatenate(probs_parts, axis=0),
        jnp.concatenate(idx_parts, axis=0),
    )
